# baseline (speedup 1.0000x reference)
import jax
import jax.numpy as jnp
from jax import lax
from jax.experimental import pallas as pl
from jax.experimental.pallas import tpu as pltpu
from jax.experimental.pallas import tpu_sc as plsc

N_NODES = 10000
N_EDGES = 320000
D = 128

NC = 2
NS = 16
NW = NC * NS
CPT = D // NW
N_PAD = 10240
FLAT = N_PAD * CPT
IB = 2000
NB = N_EDGES // IB
NG = IB // 16
GU = 5


def _process_block(buf, table, acc):
    def group(k):
        s = buf[0, pl.ds(16 * k, 16)] * CPT
        d = buf[1, pl.ds(16 * k, 16)] * CPT
        for c in range(CPT):
            g = plsc.load_gather(table, [s + c])
            plsc.addupdate_scatter(acc, [d + c], g)

    def body(i, carry):
        for u in range(GU):
            group(GU * i + u)
        return carry

    lax.fori_loop(0, NG // GU, body, 0)


def _sc_body(feat_hbm, idx_hbm, zeros_hbm, out_hbm,
             idx0, idx1, table, acc, si0, si1):
    cid = lax.axis_index("c")
    sid = lax.axis_index("s")
    wid = cid * NS + sid

    pltpu.sync_copy(feat_hbm.at[wid], table)
    pltpu.sync_copy(zeros_hbm, acc)

    pltpu.async_copy(idx_hbm.at[0], idx0, si0)

    def outer(i, carry):
        b0 = 2 * i
        b1 = b0 + 1
        pltpu.make_async_copy(idx_hbm.at[b0], idx0, si0).wait()
        pltpu.async_copy(idx_hbm.at[b1], idx1, si1)
        _process_block(idx0, table, acc)
        pltpu.make_async_copy(idx_hbm.at[b1], idx1, si1).wait()

        @pl.when(i < NB // 2 - 1)
        def _():
            pltpu.async_copy(idx_hbm.at[b0 + 2], idx0, si0)

        _process_block(idx1, table, acc)
        return carry

    lax.fori_loop(0, NB // 2, outer, 0)

    pltpu.sync_copy(acc, out_hbm.at[wid])


def _sc_aggregate(ftiles, idx):
    mesh = plsc.VectorSubcoreMesh(core_axis_name="c", subcore_axis_name="s")
    zeros = jnp.zeros((FLAT,), jnp.float32)
    return pl.kernel(
        _sc_body,
        out_type=jax.ShapeDtypeStruct((NW, FLAT), jnp.float32),
        mesh=mesh,
        compiler_params=pltpu.CompilerParams(needs_layout_passes=False),
        scratch_types=[
            pltpu.VMEM((2, IB), jnp.int32),
            pltpu.VMEM((2, IB), jnp.int32),
            pltpu.VMEM((FLAT,), jnp.float32),
            pltpu.VMEM((FLAT,), jnp.float32),
            pltpu.SemaphoreType.DMA,
            pltpu.SemaphoreType.DMA,
        ],
    )(ftiles, idx, zeros)


ROW_BLK = 1000


def _tc_body(p_ref, w_ref, b_ref, o_ref):
    o_ref[...] = (
        jnp.dot(p_ref[...], w_ref[...], preferred_element_type=jnp.float32)
        + b_ref[...]
    )


def _tc_linear(agg, W, b):
    return pl.pallas_call(
        _tc_body,
        grid=(N_NODES // ROW_BLK,),
        in_specs=[
            pl.BlockSpec((ROW_BLK, D), lambda i: (i, 0)),
            pl.BlockSpec((D, D), lambda i: (0, 0)),
            pl.BlockSpec((1, D), lambda i: (0, 0)),
        ],
        out_specs=pl.BlockSpec((ROW_BLK, D), lambda i: (i, 0)),
        out_shape=jax.ShapeDtypeStruct((N_NODES, D), jnp.float32),
    )(agg, W, b.reshape(1, D))


def kernel(features, edge_index, W, b):
    src = edge_index[0].astype(jnp.int32).reshape(NB, IB)
    dst = edge_index[1].astype(jnp.int32).reshape(NB, IB)
    idx = jnp.stack([src, dst], axis=1)
    ft = jnp.pad(features, ((0, N_PAD - N_NODES), (0, 0)))
    ftiles = ft.reshape(N_PAD, NW, CPT).transpose(1, 0, 2).reshape(NW, FLAT)
    partials = _sc_aggregate(ftiles, idx)
    agg = (partials.reshape(NW, N_PAD, CPT).transpose(1, 0, 2)
           .reshape(N_PAD, D)[:N_NODES])
    return _tc_linear(agg, W, b)


# speedup vs baseline: 4.0314x; 4.0314x over previous
import jax
import jax.numpy as jnp
from jax import lax
from jax.experimental import pallas as pl
from jax.experimental.pallas import tpu as pltpu
from jax.experimental.pallas import tpu_sc as plsc

N_NODES = 10000
N_EDGES = 320000
D = 128

NC = 2
NS = 16
NW = NC * NS
E_PER_TILE = N_EDGES // NW
CHUNK = 64
QD = 4
N_CHUNKS = 160
E_PAD = N_CHUNKS * CHUNK
NB = N_CHUNKS // QD
NM = NB // 2
N_PAD = 10240
ROWS_PER_TILE = N_PAD // NS


def _sc_body(feat_hbm, idx_hbm, zeros_hbm, out_hbm,
             idxa, idxb, r0, r1, r2, r3, acc,
             sia, sib, sg0, sg1, sg2, sg3, ss0, ss1, ss2, ss3):
    cid = lax.axis_index("c")
    sid = lax.axis_index("s")
    wid = cid * NS + sid
    row_base = sid * ROWS_PER_TILE

    rows = (r0, r1, r2, r3)
    sg = (sg0, sg1, sg2, sg3)
    ss = (ss0, ss1, ss2, ss3)

    pltpu.async_copy(idx_hbm.at[wid, 0, 0], idxa, sia)
    pltpu.sync_copy(zeros_hbm, acc.at[pl.ds(row_base, ROWS_PER_TILE)])
    plsc.subcore_barrier()

    def half(m, h, idxq, si_this, idx_pref, si_pref, last):
        pltpu.make_async_copy(idx_hbm.at[wid, m, h], idxq, si_this).wait()
        gd = [pltpu.async_copy(feat_hbm.at[idxq.at[q, 0]], rows[q], sg[q])
              for q in range(QD)]
        @pl.when(jnp.logical_not(last))
        def _():
            nm = m + h
            nh = 1 - h
            pltpu.async_copy(idx_hbm.at[wid, nm, nh], idx_pref, si_pref)

        sd = []
        for q in range(QD):
            gd[q].wait()
            sd.append(pltpu.async_copy(rows[q], acc.at[idxq.at[q, 1]],
                                       ss[q], add=True))
        for q in range(QD):
            sd[q].wait()

    def step(m, carry):
        half(m, 0, idxa, sia, idxb, sib, False)
        half(m, 1, idxb, sib, idxa, sia, m >= NM - 1)
        return carry

    lax.fori_loop(0, NM, step, 0)

    plsc.subcore_barrier()
    pltpu.sync_copy(acc.at[pl.ds(row_base, ROWS_PER_TILE)],
                    out_hbm.at[cid, pl.ds(row_base, ROWS_PER_TILE)])


def _sc_aggregate(features, idx):
    mesh = plsc.VectorSubcoreMesh(core_axis_name="c", subcore_axis_name="s")
    zeros = jnp.zeros((ROWS_PER_TILE, D), jnp.float32)
    return pl.kernel(
        _sc_body,
        out_type=jax.ShapeDtypeStruct((NC, N_PAD, D), jnp.float32),
        mesh=mesh,
        scratch_types=[
            pltpu.VMEM((QD, 2, CHUNK), jnp.int32),
            pltpu.VMEM((QD, 2, CHUNK), jnp.int32),
            pltpu.VMEM((CHUNK, D), jnp.float32),
            pltpu.VMEM((CHUNK, D), jnp.float32),
            pltpu.VMEM((CHUNK, D), jnp.float32),
            pltpu.VMEM((CHUNK, D), jnp.float32),
            pltpu.VMEM_SHARED((N_PAD, D), jnp.float32),
        ] + [pltpu.SemaphoreType.DMA] * 10,
    )(features, idx, zeros)


ROW_BLK = 1000


def _tc_body(p_ref, w_ref, b_ref, o_ref):
    agg = p_ref[0] + p_ref[1]
    o_ref[...] = (
        jnp.dot(agg, w_ref[...], preferred_element_type=jnp.float32)
        + b_ref[...]
    )


def _tc_linear(partials, W, b):
    return pl.pallas_call(
        _tc_body,
        grid=(N_NODES // ROW_BLK,),
        in_specs=[
            pl.BlockSpec((NC, ROW_BLK, D), lambda i: (0, i, 0)),
            pl.BlockSpec((D, D), lambda i: (0, 0)),
            pl.BlockSpec((1, D), lambda i: (0, 0)),
        ],
        out_specs=pl.BlockSpec((ROW_BLK, D), lambda i: (i, 0)),
        out_shape=jax.ShapeDtypeStruct((N_NODES, D), jnp.float32),
    )(partials, W, b.reshape(1, D))


def kernel(features, edge_index, W, b):
    src = edge_index[0].astype(jnp.int32).reshape(NW, E_PER_TILE)
    dst = edge_index[1].astype(jnp.int32).reshape(NW, E_PER_TILE)
    pad = E_PAD - E_PER_TILE
    pad_src = (jnp.arange(pad, dtype=jnp.int32) * 41) % N_NODES
    pad_dst = N_NODES + (jnp.arange(pad, dtype=jnp.int32) % (N_PAD - N_NODES))
    src3 = jnp.concatenate(
        [src, jnp.broadcast_to(pad_src[None], (NW, pad))], axis=1
    ).reshape(NW, N_CHUNKS, CHUNK)
    dst3 = jnp.concatenate(
        [dst, jnp.broadcast_to(pad_dst[None], (NW, pad))], axis=1
    ).reshape(NW, N_CHUNKS, CHUNK)
    idx = (jnp.stack([src3, dst3], axis=2)
           .reshape(NW, NM, 2, QD, 2, CHUNK))
    partials = _sc_aggregate(features, idx)
    return _tc_linear(partials, W, b)
